# baseline (speedup 1.0000x reference)
import functools

import jax
import jax.numpy as jnp
from jax import lax
from jax.experimental import pallas as pl
from jax.experimental.pallas import tpu as pltpu
from jax.experimental.pallas import tpu_sc as plsc

VOCAB = 100000
EMB = 300
LANES = 16
NUM_CORES = 2
NUM_SUBCORES = 16
NW = NUM_CORES * NUM_SUBCORES
BATCH = 4096
HIST = 50
ROWS_PER_W = BATCH // NW
MM_CHUNK = 2000
MM_NCHUNK = VOCAB // MM_CHUNK
MM_NBUF = 6

PR_BASE_CHUNK = 1200
PR_CHUNK = 80
PR_NBUF = 4
PR_NCHUNKS = 50
PR_GROUPS = PR_CHUNK // 16
COEF_N = 304


def _pr_body(w_hbm, wpad_hbm, p_hbm, in_buf, out_buf, coef_v, in_sems, out_sems):
    wid = lax.axis_index("s") * NUM_CORES + lax.axis_index("c")
    nk = (PR_NCHUNKS - wid + NW - 1) // NW
    pltpu.sync_copy(wpad_hbm, coef_v)
    iota = lax.iota(jnp.int32, 16)
    zeros = jnp.zeros((16,), jnp.float32)
    col0 = jnp.zeros((16,), jnp.int32)
    col1 = jnp.ones((16,), jnp.int32)

    def zinit(r, c):
        for s in range(PR_NBUF):
            out_buf[s, r] = zeros
        return c

    lax.fori_loop(0, PR_CHUNK, zinit, 0)

    for s in range(PR_NBUF - 1):
        @pl.when(s < nk)
        def _prime():
            c = PR_BASE_CHUNK + wid + s * NW
            pltpu.make_async_copy(w_hbm.at[pl.ds(c * PR_CHUNK, PR_CHUNK)],
                                  in_buf.at[s], in_sems.at[s]).start()

    def body(k, carry):
        slot = lax.rem(k, PR_NBUF)
        buf = in_buf.at[slot]
        obuf = out_buf.at[slot]
        c = PR_BASE_CHUNK + wid + k * NW
        pltpu.make_async_copy(w_hbm.at[pl.ds(c * PR_CHUNK, PR_CHUNK)],
                              buf, in_sems.at[slot]).wait()

        @pl.when(k >= PR_NBUF)
        def _owait():
            pltpu.make_async_copy(obuf, p_hbm.at[pl.ds(0, PR_CHUNK)],
                                  out_sems.at[slot]).wait()

        def tstep(t, accs):
            dbase = t * 8
            cvec = jnp.full((16,), dbase, jnp.int32)
            ca = [plsc.load_gather(coef_v, [col0, cvec + j]) for j in range(8)]
            cb = [plsc.load_gather(coef_v, [col1, cvec + j]) for j in range(8)]
            out = []
            for g in range(PR_GROUPS):
                a0, b0 = accs[2 * g], accs[2 * g + 1]
                rows = g * 16 + iota
                for j in range(8):
                    wv = plsc.load_gather(buf, [rows, cvec + j])
                    a0 = a0 + wv * ca[j]
                    b0 = b0 + wv * cb[j]
                out += [a0, b0]
            return tuple(out)

        accs = lax.fori_loop(0, COEF_N // 8, tstep, (zeros,) * (2 * PR_GROUPS))
        for g in range(PR_GROUPS):
            rows = g * 16 + iota
            plsc.store_scatter(obuf, [rows, col0], accs[2 * g])
            plsc.store_scatter(obuf, [rows, col1], accs[2 * g + 1])
        pltpu.make_async_copy(
            obuf,
            p_hbm.at[pl.ds((c - PR_BASE_CHUNK) * PR_CHUNK, PR_CHUNK)],
            out_sems.at[slot]).start()
        kn = k + PR_NBUF - 1

        @pl.when(kn < nk)
        def _refill():
            cn = PR_BASE_CHUNK + wid + kn * NW
            sn = lax.rem(kn, PR_NBUF)
            pltpu.make_async_copy(w_hbm.at[pl.ds(cn * PR_CHUNK, PR_CHUNK)],
                                  in_buf.at[sn], in_sems.at[sn]).start()

        return carry

    lax.fori_loop(0, nk, body, 0)

    def draino(s, c):
        @pl.when(s < nk)
        def _d():
            slot = lax.rem(nk - 1 - s, PR_NBUF)
            pltpu.make_async_copy(out_buf.at[slot],
                                  p_hbm.at[pl.ds(0, PR_CHUNK)],
                                  out_sems.at[slot]).wait()
        return c

    lax.fori_loop(0, PR_NBUF, draino, 0)


_project_sc = functools.partial(
    pl.kernel,
    mesh=plsc.VectorSubcoreMesh(core_axis_name="c", subcore_axis_name="s"),
    out_type=jax.ShapeDtypeStruct((PR_NCHUNKS * PR_CHUNK, LANES), jnp.float32),
    scratch_types=[
        pltpu.VMEM((PR_NBUF, PR_CHUNK, EMB), jnp.float32),
        pltpu.VMEM((PR_NBUF, PR_CHUNK, LANES), jnp.float32),
        pltpu.VMEM((2, COEF_N), jnp.float32),
        pltpu.SemaphoreType.DMA((PR_NBUF,)),
        pltpu.SemaphoreType.DMA((PR_NBUF,)),
    ],
    compiler_params=pltpu.CompilerParams(use_tc_tiling_on_sc=False,
                                         needs_layout_passes=False),
)(_pr_body)


def _mm_body(w_hbm, wo_ref, p_ref, in_buf, sems):
    i = pl.program_id(0)

    @pl.when(i == 0)
    def _prime():
        for k in range(MM_NBUF):
            pltpu.make_async_copy(w_hbm.at[pl.ds(k * MM_CHUNK, MM_CHUNK)],
                                  in_buf.at[k], sems.at[k]).start()

    b = lax.rem(i, MM_NBUF)
    pltpu.make_async_copy(w_hbm.at[pl.ds(i * MM_CHUNK, MM_CHUNK)],
                          in_buf.at[b], sems.at[b]).wait()
    p_ref[...] = jnp.dot(in_buf[b], wo_ref[...],
                         preferred_element_type=jnp.float32)
    nxt = i + MM_NBUF

    @pl.when(nxt < MM_NCHUNK)
    def _refill():
        bn = lax.rem(nxt, MM_NBUF)
        pltpu.make_async_copy(w_hbm.at[pl.ds(nxt * MM_CHUNK, MM_CHUNK)],
                              in_buf.at[bn], sems.at[bn]).start()


def _project_table(weight, w_pad):
    return pl.pallas_call(
        _mm_body,
        grid=(MM_NCHUNK,),
        in_specs=[
            pl.BlockSpec(memory_space=pl.ANY),
            pl.BlockSpec((EMB, LANES), lambda i: (0, 0)),
        ],
        out_specs=pl.BlockSpec((MM_CHUNK, LANES), lambda i: (i, 0)),
        out_shape=jax.ShapeDtypeStruct((VOCAB, LANES), jnp.float32),
        scratch_shapes=[
            pltpu.VMEM((MM_NBUF, MM_CHUNK, EMB), jnp.float32),
            pltpu.SemaphoreType.DMA((MM_NBUF,)),
        ],
    )(weight, w_pad)


def _sc_body(xt_hbm, p_hbm, out_hbm, idx_v, rows_v, out_v, sem):
    wid = lax.axis_index("s") * NUM_CORES + lax.axis_index("c")

    pltpu.sync_copy(xt_hbm.at[wid], idx_v)

    def fire(j, c):
        pltpu.make_async_copy(p_hbm.at[idx_v.at[j]], rows_v.at[j], sem).start()
        return c

    lax.fori_loop(0, HIST, fire, 0)

    def drain(j, c):
        pltpu.make_async_copy(p_hbm.at[idx_v.at[j]], rows_v.at[j], sem).wait()
        return c

    lax.fori_loop(0, HIST, drain, 0)

    def row(b, c):
        acc = rows_v[0, b]
        for j in range(1, HIST):
            acc = acc + rows_v[j, b]
        out_v[b] = acc
        return c

    lax.fori_loop(0, ROWS_PER_W, row, 0)

    pltpu.sync_copy(out_v, out_hbm.at[pl.ds(wid * ROWS_PER_W, ROWS_PER_W)])


_gather_pool = functools.partial(
    pl.kernel,
    mesh=plsc.VectorSubcoreMesh(core_axis_name="c", subcore_axis_name="s"),
    out_type=jax.ShapeDtypeStruct((BATCH, LANES), jnp.float32),
    scratch_types=[
        pltpu.VMEM((HIST, ROWS_PER_W), jnp.int32),
        pltpu.VMEM((HIST, ROWS_PER_W, LANES), jnp.float32),
        pltpu.VMEM((ROWS_PER_W, LANES), jnp.float32),
        pltpu.SemaphoreType.DMA,
    ],
    compiler_params=pltpu.CompilerParams(use_tc_tiling_on_sc=False),
)(_sc_body)


def kernel(x, weight, W_out):
    n_out = W_out.shape[0]
    w_pad = jnp.zeros((EMB, LANES), jnp.float32)
    w_pad = w_pad.at[:, :n_out].set(W_out.T.astype(jnp.float32) * (1.0 / HIST))
    p = _project_table(weight, w_pad)
    w_coef = jnp.zeros((2, COEF_N), jnp.float32)
    w_coef = w_coef.at[:n_out, :EMB].set(W_out.astype(jnp.float32) * (1.0 / HIST))
    p2 = _project_sc(weight, w_coef)
    xt = x.astype(jnp.int32).reshape(NW, ROWS_PER_W, HIST).transpose(0, 2, 1)
    out16 = _gather_pool(xt, p)
    return out16[:, :n_out] + p2.sum() * 1e-30


# speedup vs baseline: 2.1226x; 2.1226x over previous
import functools

import jax
import jax.numpy as jnp
from jax import lax
from jax.experimental import pallas as pl
from jax.experimental.pallas import tpu as pltpu
from jax.experimental.pallas import tpu_sc as plsc

VOCAB = 100000
EMB = 300
LANES = 16
NUM_CORES = 2
NUM_SUBCORES = 16
NW = NUM_CORES * NUM_SUBCORES
BATCH = 4096
HIST = 50
ROWS_PER_W = BATCH // NW
MM_CHUNK = 2000
MM_NCHUNK = VOCAB // MM_CHUNK
MM_NBUF = 6


def _mm_body(w_hbm, wo_ref, p_ref, in_buf, sems):
    i = pl.program_id(0)

    @pl.when(i == 0)
    def _prime():
        for k in range(MM_NBUF):
            pltpu.make_async_copy(w_hbm.at[pl.ds(k * MM_CHUNK, MM_CHUNK)],
                                  in_buf.at[k], sems.at[k]).start()

    b = lax.rem(i, MM_NBUF)
    pltpu.make_async_copy(w_hbm.at[pl.ds(i * MM_CHUNK, MM_CHUNK)],
                          in_buf.at[b], sems.at[b]).wait()
    p_ref[...] = jnp.dot(in_buf[b], wo_ref[...],
                         preferred_element_type=jnp.float32)
    nxt = i + MM_NBUF

    @pl.when(nxt < MM_NCHUNK)
    def _refill():
        bn = lax.rem(nxt, MM_NBUF)
        pltpu.make_async_copy(w_hbm.at[pl.ds(nxt * MM_CHUNK, MM_CHUNK)],
                              in_buf.at[bn], sems.at[bn]).start()


def _project_table(weight, w_pad):
    return pl.pallas_call(
        _mm_body,
        grid=(MM_NCHUNK,),
        in_specs=[
            pl.BlockSpec(memory_space=pl.ANY),
            pl.BlockSpec((EMB, LANES), lambda i: (0, 0)),
        ],
        out_specs=pl.BlockSpec((MM_CHUNK, LANES), lambda i: (i, 0)),
        out_shape=jax.ShapeDtypeStruct((VOCAB, LANES), jnp.float32),
        scratch_shapes=[
            pltpu.VMEM((MM_NBUF, MM_CHUNK, EMB), jnp.float32),
            pltpu.SemaphoreType.DMA((MM_NBUF,)),
        ],
    )(weight, w_pad)


def _sc_body(xt_hbm, p_hbm, out_hbm, idx_v, rows_v, out_v, sem):
    wid = lax.axis_index("s") * NUM_CORES + lax.axis_index("c")

    pltpu.sync_copy(xt_hbm.at[wid], idx_v)

    def fire(j, c):
        pltpu.make_async_copy(p_hbm.at[idx_v.at[j]], rows_v.at[j], sem).start()
        return c

    lax.fori_loop(0, HIST, fire, 0)

    def drain(j, c):
        pltpu.make_async_copy(p_hbm.at[idx_v.at[j]], rows_v.at[j], sem).wait()
        return c

    lax.fori_loop(0, HIST, drain, 0)

    def row(b, c):
        acc = rows_v[0, b]
        for j in range(1, HIST):
            acc = acc + rows_v[j, b]
        out_v[b] = acc
        return c

    lax.fori_loop(0, ROWS_PER_W, row, 0)

    pltpu.sync_copy(out_v, out_hbm.at[pl.ds(wid * ROWS_PER_W, ROWS_PER_W)])


_gather_pool = functools.partial(
    pl.kernel,
    mesh=plsc.VectorSubcoreMesh(core_axis_name="c", subcore_axis_name="s"),
    out_type=jax.ShapeDtypeStruct((BATCH, LANES), jnp.float32),
    scratch_types=[
        pltpu.VMEM((HIST, ROWS_PER_W), jnp.int32),
        pltpu.VMEM((HIST, ROWS_PER_W, LANES), jnp.float32),
        pltpu.VMEM((ROWS_PER_W, LANES), jnp.float32),
        pltpu.SemaphoreType.DMA,
    ],
    compiler_params=pltpu.CompilerParams(use_tc_tiling_on_sc=False),
)(_sc_body)


def kernel(x, weight, W_out):
    n_out = W_out.shape[0]
    w_pad = jnp.zeros((EMB, LANES), jnp.float32)
    w_pad = w_pad.at[:, :n_out].set(W_out.T.astype(jnp.float32) * (1.0 / HIST))
    p = _project_table(weight, w_pad)
    xt = x.astype(jnp.int32).reshape(NW, ROWS_PER_W, HIST).transpose(0, 2, 1)
    out16 = _gather_pool(xt, p)
    return out16[:, :n_out]
